# baseline (speedup 1.0000x reference)
import numpy as np

import jax
import jax.numpy as jnp
from jax.experimental import pallas as pl
from jax.experimental.pallas import tpu as pltpu

_NMS_THRESH = 0.7
_NUM_PRE = 6000
_NUM_POST = 300
_MIN_SIZE = 16.0
_STRIDE = 16
_RATIOS = (0.5, 1.0, 2.0)
_SCALES = (8, 16, 32)
_A = 9
_FH = 50
_FW = 50
_NB = _FH * _FW * _A
_PLANE = (176, 128)
_PAD_N = 176 * 128 - _NB

_HP = _FH + 2
_NPIX = _HP * _HP
_OFFS = tuple((dh - 1) * _HP + (dw - 1) for dh in range(3) for dw in range(3))
_SHIFT = 53


def _base_anchors():
    py = px = _STRIDE / 2.0
    ab = np.zeros((len(_RATIOS) * len(_SCALES), 4), dtype=np.float32)
    for i, r in enumerate(_RATIOS):
        for j, s in enumerate(_SCALES):
            h = _STRIDE * s * np.sqrt(r)
            w = _STRIDE * s * np.sqrt(1.0 / r)
            k = i * len(_SCALES) + j
            ab[k] = [py - h / 2.0, px - w / 2.0, py + h / 2.0, px + w / 2.0]
    return ab


def _anchors_np():
    base = _base_anchors()
    sy = np.arange(64, dtype=np.float32) * _STRIDE
    sx = np.arange(64, dtype=np.float32) * _STRIDE
    yy, xx = np.meshgrid(sy, sx, indexing='ij')
    shifts = np.stack([yy, xx, yy, xx], axis=-1)
    allb = shifts[:, :, None, :] + base[None, None, :, :]
    return allb[:_FH, :_FW].reshape(-1, 4)


def _k1(x_ref, w9_ref, b1_ref, w2_ref, b2_ref, o_ref):
    cols = jnp.concatenate(
        [x_ref[pl.ds(_SHIFT + off, _NPIX), :] for off in _OFFS], axis=1)
    acc = jnp.dot(cols, w9_ref[:], preferred_element_type=jnp.float32)
    acc = acc + b1_ref[:]
    acc = jnp.where(acc >= 0.0, acc, 0.01 * acc)
    o_ref[:] = jnp.dot(acc.astype(jnp.bfloat16), w2_ref[:],
                       preferred_element_type=jnp.float32) + b2_ref[:]


def _k2(dy_ref, dx_ref, dhh_ref, dww_ref, s0_ref, s1_ref,
        ha_ref, wa_ref, cya_ref, cxa_ref, img_ref, out_ref,
        y1s, x1s, y2s, x2s, ars):
    f32 = jnp.float32
    NEG = jnp.float32(-jnp.inf)
    idx = (jax.lax.broadcasted_iota(jnp.int32, _PLANE, 0) * 128
           + jax.lax.broadcasted_iota(jnp.int32, _PLANE, 1))
    real = idx < _NB

    s0 = s0_ref[:]
    s1 = s1_ref[:]
    m = jnp.maximum(s0, s1)
    e0 = jnp.exp(s0 - m)
    e1 = jnp.exp(s1 - m)
    fg = e1 / (e0 + e1)

    ha = ha_ref[:]
    wa = wa_ref[:]
    ncy = (0.1 * dy_ref[:]) * ha + cya_ref[:]
    ncx = (0.1 * dx_ref[:]) * wa + cxa_ref[:]
    nh = jnp.exp(0.2 * dhh_ref[:]) * ha
    nw = jnp.exp(0.2 * dww_ref[:]) * wa
    img_h = img_ref[0, 0]
    img_w = img_ref[0, 1]
    y1 = jnp.minimum(jnp.maximum(ncy - 0.5 * nh, 0.0), img_h)
    x1 = jnp.minimum(jnp.maximum(ncx - 0.5 * nw, 0.0), img_w)
    y2 = jnp.minimum(jnp.maximum(ncy + 0.5 * nh, 0.0), img_h)
    x2 = jnp.minimum(jnp.maximum(ncx + 0.5 * nw, 0.0), img_w)
    hh = y2 - y1
    ww = x2 - x1
    valid = (hh >= _MIN_SIZE) & (ww >= _MIN_SIZE)
    area = hh * ww

    bits = jax.lax.bitcast_convert_type(fg, jnp.int32)
    key = jnp.where(real, bits, jnp.int32(-1))

    def bs_body(_, carry):
        lo, hi = carry
        mid = lo + (hi - lo) // 2
        c = jnp.sum((key >= mid).astype(jnp.int32))
        big = c >= _NUM_PRE
        return (jnp.where(big, mid, lo), jnp.where(big, hi, mid))

    lo0 = jnp.int32(0)
    hi0 = jnp.int32(0x3F800001)
    lo, hi = jax.lax.fori_loop(0, 31, bs_body, (lo0, hi0))
    T = lo
    c_gt = jnp.sum((key > T).astype(jnp.int32))
    r = _NUM_PRE - c_gt

    eq = key == T

    def bs2_body(_, carry):
        lo2, hi2 = carry
        mid = lo2 + (hi2 - lo2) // 2
        c = jnp.sum((eq & (idx < mid)).astype(jnp.int32))
        small = c < r
        return (jnp.where(small, mid, lo2), jnp.where(small, hi2, mid))

    lo2, hi2 = jax.lax.fori_loop(0, 16, bs2_body, (jnp.int32(0), jnp.int32(_NB)))
    kcut = jnp.where(r > 0, hi2, jnp.int32(0))
    member = (key > T) | (eq & (idx < kcut))

    msc0 = jnp.where(member & valid, fg, NEG)

    y1s[...] = y1
    x1s[...] = x1
    y2s[...] = y2
    x2s[...] = x2
    ars[...] = area

    lane = jax.lax.broadcasted_iota(jnp.int32, (1, 128), 1)
    BIG = jnp.int32(2 ** 30)
    zero = jnp.float32(0.0)

    def nms_body(i, msc):
        mx0 = jnp.max(msc, axis=0, keepdims=True)
        i0 = jnp.min(jnp.where(msc == mx0, idx, BIG), axis=0, keepdims=True)
        v, ix = mx0, i0
        for k in (1, 2, 4, 8, 16, 32, 64):
            rv = pltpu.roll(v, k, 1)
            ri = pltpu.roll(ix, k, 1)
            take = (rv > v) | ((rv == v) & (ri < ix))
            v = jnp.where(take, rv, v)
            ix = jnp.where(take, ri, ix)
        validv = v > NEG
        pickv = jnp.where(validv, ix, jnp.int32(0))
        cell = idx == pickv
        y1v = y1s[...]
        x1v = x1s[...]
        y2v = y2s[...]
        x2v = x2s[...]
        arv = ars[...]
        ry1 = jnp.sum(jnp.where(cell, y1v, zero), axis=0, keepdims=True)
        rx1 = jnp.sum(jnp.where(cell, x1v, zero), axis=0, keepdims=True)
        ry2 = jnp.sum(jnp.where(cell, y2v, zero), axis=0, keepdims=True)
        rx2 = jnp.sum(jnp.where(cell, x2v, zero), axis=0, keepdims=True)
        rar = jnp.sum(jnp.where(cell, arv, zero), axis=0, keepdims=True)
        m8 = jnp.concatenate([ry1, rx1, ry2, rx2, rar,
                              jnp.zeros((3, 128), jnp.float32)], axis=0)
        b8 = m8
        for k in (1, 2, 4, 8, 16, 32, 64):
            b8 = b8 + pltpu.roll(b8, k, 1)
        by1 = b8[0:1, :]
        bx1 = b8[1:2, :]
        by2 = b8[2:3, :]
        bx2 = b8[3:4, :]
        barea = b8[4:5, :]
        yy1 = jnp.maximum(by1, y1v)
        xx1 = jnp.maximum(bx1, x1v)
        yy2 = jnp.minimum(by2, y2v)
        xx2 = jnp.minimum(bx2, x2v)
        inter = jnp.maximum(yy2 - yy1, 0.0) * jnp.maximum(xx2 - xx1, 0.0)
        iou = inter / (barea + arv - inter + 1e-9)
        msc = jnp.where(validv & (iou > _NMS_THRESH), NEG, msc)
        row = jnp.where(lane == 0, by1,
              jnp.where(lane == 1, bx1,
              jnp.where(lane == 2, by2,
              jnp.where(lane == 3, bx2, zero))))
        row = jnp.where(validv, row, jnp.zeros_like(row))
        out_ref[pl.ds(i, 1), :] = row
        return msc

    jax.lax.fori_loop(0, _NUM_POST, nms_body, msc0)


def _plane_f32(v):
    return jnp.pad(v, (0, _PAD_N)).reshape(_PLANE)


def kernel(x, img_size, conv_w, conv_b, score_w, score_b, loc_w, loc_b):
    f32 = jnp.float32
    xt = jnp.transpose(x[0], (1, 2, 0))
    xp = jnp.pad(xt, ((1, 1), (1, 1), (0, 0)))
    xbig = jnp.pad(xp.reshape(_NPIX, 512),
                   ((_SHIFT, _SHIFT), (0, 0))).astype(jnp.bfloat16)
    w9 = jnp.transpose(conv_w, (2, 3, 1, 0)).reshape(4608, 512).astype(jnp.bfloat16)
    lw = jnp.transpose(loc_w[:, :, 0, 0])
    sw = jnp.transpose(score_w[:, :, 0, 0])
    w2 = jnp.pad(jnp.concatenate([lw, sw], axis=1),
                 ((0, 0), (0, 74))).astype(jnp.bfloat16)
    b2 = jnp.pad(jnp.concatenate([loc_b, score_b]), (0, 74)).reshape(1, 128)
    b1 = conv_b.reshape(1, 512)

    out2 = pl.pallas_call(
        _k1,
        out_shape=jax.ShapeDtypeStruct((_NPIX, 128), f32),
    )(xbig, w9, b1, w2, b2)

    o = out2.reshape(_HP, _HP, 128)[1:51, 1:51, :].reshape(_FH * _FW, 128)
    rpn_locs = o[:, :36].reshape(_NB, 4)
    rpn_scores = o[:, 36:54].reshape(_NB, 2)

    anc = _anchors_np()
    ha_np = anc[:, 2] - anc[:, 0]
    wa_np = anc[:, 3] - anc[:, 1]
    cya_np = anc[:, 0] + 0.5 * ha_np
    cxa_np = anc[:, 1] + 0.5 * wa_np
    anchors = jnp.asarray(anc)

    dy = _plane_f32(rpn_locs[:, 0])
    dx = _plane_f32(rpn_locs[:, 1])
    dhh = _plane_f32(rpn_locs[:, 2])
    dww = _plane_f32(rpn_locs[:, 3])
    s0 = _plane_f32(rpn_scores[:, 0])
    s1 = _plane_f32(rpn_scores[:, 1])
    ha = _plane_f32(jnp.asarray(ha_np))
    wa = _plane_f32(jnp.asarray(wa_np))
    cya = _plane_f32(jnp.asarray(cya_np))
    cxa = _plane_f32(jnp.asarray(cxa_np))
    img = img_size.astype(f32).reshape(1, 2)

    rois_pad = pl.pallas_call(
        _k2,
        out_shape=jax.ShapeDtypeStruct((_NUM_POST + 4, 128), f32),
        scratch_shapes=[pltpu.VMEM(_PLANE, f32)] * 5,
    )(dy, dx, dhh, dww, s0, s1, ha, wa, cya, cxa, img)

    rois = rois_pad[:_NUM_POST, :4]
    return rpn_locs, rpn_scores, rois, anchors


# speedup vs baseline: 1.6806x; 1.6806x over previous
import numpy as np

import jax
import jax.numpy as jnp
from jax.experimental import pallas as pl
from jax.experimental.pallas import tpu as pltpu

_NMS_THRESH = 0.7
_NUM_PRE = 6000
_NUM_POST = 300
_MIN_SIZE = 16.0
_STRIDE = 16
_RATIOS = (0.5, 1.0, 2.0)
_SCALES = (8, 16, 32)
_A = 9
_FH = 50
_FW = 50
_NB = _FH * _FW * _A
_PLANE = (176, 128)
_PAD_N = 176 * 128 - _NB

_HP = _FH + 2
_NPIX = _HP * _HP
_OFFS = tuple((dh - 1) * _HP + (dw - 1) for dh in range(3) for dw in range(3))
_SHIFT = 53


def _base_anchors():
    py = px = _STRIDE / 2.0
    ab = np.zeros((len(_RATIOS) * len(_SCALES), 4), dtype=np.float32)
    for i, r in enumerate(_RATIOS):
        for j, s in enumerate(_SCALES):
            h = _STRIDE * s * np.sqrt(r)
            w = _STRIDE * s * np.sqrt(1.0 / r)
            k = i * len(_SCALES) + j
            ab[k] = [py - h / 2.0, px - w / 2.0, py + h / 2.0, px + w / 2.0]
    return ab


def _anchors_np():
    base = _base_anchors()
    sy = np.arange(64, dtype=np.float32) * _STRIDE
    sx = np.arange(64, dtype=np.float32) * _STRIDE
    yy, xx = np.meshgrid(sy, sx, indexing='ij')
    shifts = np.stack([yy, xx, yy, xx], axis=-1)
    allb = shifts[:, :, None, :] + base[None, None, :, :]
    return allb[:_FH, :_FW].reshape(-1, 4)


def _k1(x_ref, w9_ref, b1_ref, w2_ref, b2_ref, o_ref):
    cols = jnp.concatenate(
        [x_ref[pl.ds(_SHIFT + off, _NPIX), :] for off in _OFFS], axis=1)
    acc = jnp.dot(cols, w9_ref[:], preferred_element_type=jnp.float32)
    acc = acc + b1_ref[:]
    acc = jnp.where(acc >= 0.0, acc, 0.01 * acc)
    o_ref[:] = jnp.dot(acc.astype(jnp.bfloat16), w2_ref[:],
                       preferred_element_type=jnp.float32) + b2_ref[:]


def _k2(dy_ref, dx_ref, dhh_ref, dww_ref, s0_ref, s1_ref,
        ha_ref, wa_ref, cya_ref, cxa_ref, img_ref, out_ref,
        y1s, x1s, y2s, x2s, ars):
    f32 = jnp.float32
    NEG = jnp.float32(-jnp.inf)
    idx = (jax.lax.broadcasted_iota(jnp.int32, _PLANE, 0) * 128
           + jax.lax.broadcasted_iota(jnp.int32, _PLANE, 1))
    real = idx < _NB

    s0 = s0_ref[:]
    s1 = s1_ref[:]
    m = jnp.maximum(s0, s1)
    e0 = jnp.exp(s0 - m)
    e1 = jnp.exp(s1 - m)
    fg = e1 / (e0 + e1)

    ha = ha_ref[:]
    wa = wa_ref[:]
    ncy = (0.1 * dy_ref[:]) * ha + cya_ref[:]
    ncx = (0.1 * dx_ref[:]) * wa + cxa_ref[:]
    nh = jnp.exp(0.2 * dhh_ref[:]) * ha
    nw = jnp.exp(0.2 * dww_ref[:]) * wa
    img_h = img_ref[0, 0]
    img_w = img_ref[0, 1]
    y1 = jnp.minimum(jnp.maximum(ncy - 0.5 * nh, 0.0), img_h)
    x1 = jnp.minimum(jnp.maximum(ncx - 0.5 * nw, 0.0), img_w)
    y2 = jnp.minimum(jnp.maximum(ncy + 0.5 * nh, 0.0), img_h)
    x2 = jnp.minimum(jnp.maximum(ncx + 0.5 * nw, 0.0), img_w)
    hh = y2 - y1
    ww = x2 - x1
    valid = (hh >= _MIN_SIZE) & (ww >= _MIN_SIZE)
    area = hh * ww

    bits = jax.lax.bitcast_convert_type(fg, jnp.int32)
    key = jnp.where(real, bits, jnp.int32(-1))

    def bs_body(_, carry):
        lo, hi = carry
        mid = lo + (hi - lo) // 2
        c = jnp.sum((key >= mid).astype(jnp.int32))
        big = c >= _NUM_PRE
        return (jnp.where(big, mid, lo), jnp.where(big, hi, mid))

    lo0 = jnp.int32(0)
    hi0 = jnp.int32(0x3F800001)
    lo, hi = jax.lax.fori_loop(0, 31, bs_body, (lo0, hi0))
    T = lo
    c_gt = jnp.sum((key > T).astype(jnp.int32))
    r = _NUM_PRE - c_gt

    eq = key == T

    def bs2_body(_, carry):
        lo2, hi2 = carry
        mid = lo2 + (hi2 - lo2) // 2
        c = jnp.sum((eq & (idx < mid)).astype(jnp.int32))
        small = c < r
        return (jnp.where(small, mid, lo2), jnp.where(small, hi2, mid))

    lo2, hi2 = jax.lax.fori_loop(0, 16, bs2_body, (jnp.int32(0), jnp.int32(_NB)))
    kcut = jnp.where(r > 0, hi2, jnp.int32(0))
    member = (key > T) | (eq & (idx < kcut))

    msc0 = jnp.where(member & valid, fg, NEG)

    y1s[...] = y1
    x1s[...] = x1
    y2s[...] = y2
    x2s[...] = x2
    ars[...] = area

    lane = jax.lax.broadcasted_iota(jnp.int32, (1, 128), 1)
    BIG = jnp.int32(2 ** 30)
    zero = jnp.float32(0.0)

    def nms_body(i, msc):
        mx0 = jnp.max(msc, axis=0, keepdims=True)
        i0 = jnp.min(jnp.where(msc == mx0, idx, BIG), axis=0, keepdims=True)
        mx = jnp.max(mx0)
        pick = jnp.min(jnp.where(mx0 == mx, i0, BIG))
        validp = mx > NEG
        pick = jnp.where(validp, pick, jnp.int32(0))
        prow = jax.lax.shift_right_logical(pick, 7)
        sel = lane == (pick & 127)
        by1 = jnp.sum(jnp.where(sel, y1s[pl.ds(prow, 1), :], zero))
        bx1 = jnp.sum(jnp.where(sel, x1s[pl.ds(prow, 1), :], zero))
        by2 = jnp.sum(jnp.where(sel, y2s[pl.ds(prow, 1), :], zero))
        bx2 = jnp.sum(jnp.where(sel, x2s[pl.ds(prow, 1), :], zero))
        barea = jnp.sum(jnp.where(sel, ars[pl.ds(prow, 1), :], zero))
        yy1 = jnp.maximum(by1, y1s[...])
        xx1 = jnp.maximum(bx1, x1s[...])
        yy2 = jnp.minimum(by2, y2s[...])
        xx2 = jnp.minimum(bx2, x2s[...])
        inter = jnp.maximum(yy2 - yy1, 0.0) * jnp.maximum(xx2 - xx1, 0.0)
        iou = inter / (barea + ars[...] - inter + 1e-9)
        msc = jnp.where(validp & (iou > _NMS_THRESH), NEG, msc)
        row = jnp.where(lane == 0, by1,
              jnp.where(lane == 1, bx1,
              jnp.where(lane == 2, by2,
              jnp.where(lane == 3, bx2, zero))))
        row = jnp.where(validp, row, jnp.zeros_like(row))
        out_ref[pl.ds(i, 1), :] = row
        return msc

    jax.lax.fori_loop(0, _NUM_POST, nms_body, msc0)


def _plane_f32(v):
    return jnp.pad(v, (0, _PAD_N)).reshape(_PLANE)


def kernel(x, img_size, conv_w, conv_b, score_w, score_b, loc_w, loc_b):
    f32 = jnp.float32
    xt = jnp.transpose(x[0], (1, 2, 0))
    xp = jnp.pad(xt, ((1, 1), (1, 1), (0, 0)))
    xbig = jnp.pad(xp.reshape(_NPIX, 512),
                   ((_SHIFT, _SHIFT), (0, 0))).astype(jnp.bfloat16)
    w9 = jnp.transpose(conv_w, (2, 3, 1, 0)).reshape(4608, 512).astype(jnp.bfloat16)
    lw = jnp.transpose(loc_w[:, :, 0, 0])
    sw = jnp.transpose(score_w[:, :, 0, 0])
    w2 = jnp.pad(jnp.concatenate([lw, sw], axis=1),
                 ((0, 0), (0, 74))).astype(jnp.bfloat16)
    b2 = jnp.pad(jnp.concatenate([loc_b, score_b]), (0, 74)).reshape(1, 128)
    b1 = conv_b.reshape(1, 512)

    out2 = pl.pallas_call(
        _k1,
        out_shape=jax.ShapeDtypeStruct((_NPIX, 128), f32),
    )(xbig, w9, b1, w2, b2)

    o = out2.reshape(_HP, _HP, 128)[1:51, 1:51, :].reshape(_FH * _FW, 128)
    rpn_locs = o[:, :36].reshape(_NB, 4)
    rpn_scores = o[:, 36:54].reshape(_NB, 2)

    anc = _anchors_np()
    ha_np = anc[:, 2] - anc[:, 0]
    wa_np = anc[:, 3] - anc[:, 1]
    cya_np = anc[:, 0] + 0.5 * ha_np
    cxa_np = anc[:, 1] + 0.5 * wa_np
    anchors = jnp.asarray(anc)

    dy = _plane_f32(rpn_locs[:, 0])
    dx = _plane_f32(rpn_locs[:, 1])
    dhh = _plane_f32(rpn_locs[:, 2])
    dww = _plane_f32(rpn_locs[:, 3])
    s0 = _plane_f32(rpn_scores[:, 0])
    s1 = _plane_f32(rpn_scores[:, 1])
    ha = _plane_f32(jnp.asarray(ha_np))
    wa = _plane_f32(jnp.asarray(wa_np))
    cya = _plane_f32(jnp.asarray(cya_np))
    cxa = _plane_f32(jnp.asarray(cxa_np))
    img = img_size.astype(f32).reshape(1, 2)

    rois_pad = pl.pallas_call(
        _k2,
        out_shape=jax.ShapeDtypeStruct((_NUM_POST + 4, 128), f32),
        scratch_shapes=[pltpu.VMEM(_PLANE, f32)] * 5,
    )(dy, dx, dhh, dww, s0, s1, ha, wa, cya, cxa, img)

    rois = rois_pad[:_NUM_POST, :4]
    return rpn_locs, rpn_scores, rois, anchors
